# Initial kernel scaffold; baseline (speedup 1.0000x reference)
import jax
import jax.numpy as jnp
from jax.experimental import pallas as pl


def kernel(x, e, edges, template, W1, b1, W2, b2, W3, b3, g1, be1, g2, be2, g3, be3, eW1, eb1, eW2, eb2, eW3, eb3, eW4, eb4):
    raise NotImplementedError("write your pallas kernel here")


# speedup vs baseline: 37.6915x; 37.6915x over previous
import functools

import jax
import jax.numpy as jnp
from jax import lax
from jax.experimental import pallas as pl
from jax.experimental.pallas import tpu as pltpu
from jax.experimental.pallas import tpu_sc as plsc

B = 4
DIM = 3
N = 10000
E_T = 30000
E_D = 30000
D2 = 6

NC = 2
NS = 16
NW = NC * NS
VL = 16

N_P = 10240
NR = N_P // NS
E_P = 32768
EC = E_P // NS
EPT = 7680
EP_D = 8 * EPT
XN = 30720

f32 = jnp.float32
i32 = jnp.int32


def _iota():
    return lax.iota(i32, VL)


def _fastrsqrt(a):
    i = plsc.bitcast(a, i32)
    i = jnp.int32(0x5F3759DF) - lax.shift_right_logical(i, 1)
    y = plsc.bitcast(i, f32)
    for _ in range(3):
        y = y * (1.5 - 0.5 * a * y * y)
    return y


def _scalar_vec(pairs):
    it = _iota()
    v = jnp.zeros((VL,), f32)
    for lane, s in pairs:
        v = jnp.where(it == lane, jnp.full((VL,), s, dtype=f32), v)
    return v


def _s1_body(src2d, dst2d, valfl, tmpl, e_flat, x_flat, zrow,
             degv_o, gb_o, cpart_o, dinv_o, dinv2_o, normv_o,
             ideg_s, degv_s, gb_s, dinv_s,
             buf30k, echk, i16a, i16b, fv, msg,
             d1, dy, dy2, nbuf, s128, sem):
    cid = lax.axis_index("c")
    sid = lax.axis_index("s")
    wid = cid * NS + sid
    r0 = sid * NR
    eb = sid * EC

    pltpu.sync_copy(zrow, ideg_s.at[pl.ds(r0, NR)])
    pltpu.sync_copy(zrow, degv_s.at[pl.ds(r0, NR)])
    for dim in range(DIM):
        pltpu.sync_copy(zrow, gb_s.at[pl.ds(dim * N_P + r0, NR)])

    def z128(i, c):
        s128[pl.ds(i * VL, VL)] = jnp.zeros((VL,), f32)
        return c
    lax.fori_loop(0, 8, z128, 0)
    plsc.subcore_barrier()

    pltpu.sync_copy(dst2d.at[pl.ds(sid * 16, 16)], i16a)
    pltpu.sync_copy(src2d.at[pl.ds(sid * 16, 16)], i16b)
    pltpu.sync_copy(valfl.at[pl.ds(eb, EC)], fv)

    hs = [pltpu.async_copy(fv.at[pl.ds(j * 128, 128)],
                           ideg_s.at[i16a.at[j]], sem, add=True)
          for j in range(16)]
    for h in hs:
        h.wait()

    @pl.when(cid == 0)
    def _():
        hs0 = [pltpu.async_copy(fv.at[pl.ds(j * 128, 128)],
                                degv_s.at[i16a.at[j]], sem, add=True)
               for j in range(16)]
        hs1 = [pltpu.async_copy(fv.at[pl.ds(j * 128, 128)],
                                degv_s.at[i16b.at[j]], sem, add=True)
               for j in range(16)]
        for h in hs0 + hs1:
            h.wait()

    @pl.when(cid == 1)
    def _():
        pltpu.sync_copy(tmpl, buf30k)

        def build(sign):
            def bodyf(j, c):
                jf = jnp.broadcast_to(lax.shift_right_logical(j, 3), (VL,))
                posv = _iota() + (j & 7) * VL
                sv = plsc.load_gather(i16b, [jf, posv])
                dv = plsc.load_gather(i16a, [jf, posv])
                vf = fv[pl.ds(j * VL, VL)]
                for dim in range(DIM):
                    a = plsc.load_gather(buf30k, [sv + dim * N])
                    b = plsc.load_gather(buf30k, [dv + dim * N])
                    cq = 0.5 * (a + b)
                    tq = b - a
                    msg[pl.ds(dim * EC + j * VL, VL)] = (
                        (cq + sign * 2.0 * tq) * vf)
                return c
            lax.fori_loop(0, EC // VL, bodyf, 0)

        build(-1.0)
        hs2 = [pltpu.async_copy(msg.at[pl.ds(dim * EC + j * 128, 128)],
                                gb_s.at[pl.ds(dim * N_P, N_P)].at[i16b.at[j]],
                                sem, add=True)
               for dim in range(DIM) for j in range(16)]
        for h in hs2:
            h.wait()
        build(1.0)
        hs3 = [pltpu.async_copy(msg.at[pl.ds(dim * EC + j * 128, 128)],
                                gb_s.at[pl.ds(dim * N_P, N_P)].at[i16a.at[j]],
                                sem, add=True)
               for dim in range(DIM) for j in range(16)]
        for h in hs3:
            h.wait()

    smp = lax.shift_right_logical(wid, 3)
    part = wid & 7
    pltpu.sync_copy(e_flat.at[pl.ds(smp * EP_D + part * EPT, EPT)], echk)
    pltpu.sync_copy(x_flat.at[pl.ds(smp * XN, XN)], buf30k)

    def cbody(i, carry):
        c0, c1, c2 = carry
        ev = echk[pl.ds(i * VL, VL)]
        posv = _iota() + part * EPT + i * VL
        mb = posv < (2 * E_D)
        g0 = plsc.load_gather(buf30k, [ev])
        g1 = plsc.load_gather(buf30k, [ev + N])
        g2 = plsc.load_gather(buf30k, [ev + 2 * N])
        zero = jnp.zeros((VL,), f32)
        return (c0 + jnp.where(mb, g0, zero),
                c1 + jnp.where(mb, g1, zero),
                c2 + jnp.where(mb, g2, zero))

    z = jnp.zeros((VL,), f32)
    c0, c1, c2 = lax.fori_loop(0, EPT // VL, cbody, (z, z, z))
    s128[pl.ds(0, VL)] = _scalar_vec([(0, jnp.sum(c0)), (1, jnp.sum(c1)),
                                      (2, jnp.sum(c2))])
    pltpu.sync_copy(s128, cpart_o.at[pl.ds(wid * 128, 128)])
    plsc.subcore_barrier()

    pltpu.sync_copy(ideg_s.at[pl.ds(r0, NR)], d1)

    def dbody(i, c):
        a = d1[pl.ds(i * VL, VL)] + 1.0
        y = _fastrsqrt(a)
        dy[pl.ds(i * VL, VL)] = y
        dy2[pl.ds(i * VL, VL)] = y * y
        return c
    lax.fori_loop(0, NR // VL, dbody, 0)

    pltpu.sync_copy(dy, dinv_s.at[pl.ds(r0, NR)])

    @pl.when(cid == 0)
    def _():
        pltpu.sync_copy(dy, dinv_o.at[pl.ds(r0, NR)])
        pltpu.sync_copy(dy2, dinv2_o.at[pl.ds(r0, NR)])
        pltpu.sync_copy(degv_s.at[pl.ds(r0, NR)], d1)
        pltpu.sync_copy(d1, degv_o.at[pl.ds(r0, NR)])

    @pl.when(cid == 1)
    def _():
        for dim in range(DIM):
            pltpu.sync_copy(gb_s.at[pl.ds(dim * N_P + r0, NR)], d1)
            pltpu.sync_copy(d1, gb_o.at[pl.ds(dim * N_P + r0, NR)])

    plsc.subcore_barrier()

    pltpu.sync_copy(dinv_s, buf30k.at[pl.ds(0, N_P)])
    nb = wid * 1024
    pltpu.sync_copy(src2d.at[pl.ds(wid * 8, 8)], i16b.at[pl.ds(0, 8)])
    pltpu.sync_copy(dst2d.at[pl.ds(wid * 8, 8)], i16a.at[pl.ds(0, 8)])
    pltpu.sync_copy(valfl.at[pl.ds(nb, 1024)], fv.at[pl.ds(0, 1024)])

    def nbody(j, c):
        jf = jnp.broadcast_to(lax.shift_right_logical(j, 3), (VL,))
        posv = _iota() + (j & 7) * VL
        sv = plsc.load_gather(i16b, [jf, posv])
        dv = plsc.load_gather(i16a, [jf, posv])
        vf = fv[pl.ds(j * VL, VL)]
        nv = plsc.load_gather(buf30k, [sv]) * plsc.load_gather(buf30k, [dv])
        nbuf[pl.ds(j * VL, VL)] = nv * vf
        return c
    lax.fori_loop(0, 1024 // VL, nbody, 0)
    pltpu.sync_copy(nbuf, normv_o.at[pl.ds(nb, 1024)])


def _make_s1():
    mesh = plsc.VectorSubcoreMesh(core_axis_name="c", subcore_axis_name="s")
    return pl.kernel(
        _s1_body,
        compiler_params=pltpu.CompilerParams(needs_layout_passes=False),
        out_type=[
            jax.ShapeDtypeStruct((N_P,), f32),
            jax.ShapeDtypeStruct((DIM * N_P,), f32),
            jax.ShapeDtypeStruct((NW * 128,), f32),
            jax.ShapeDtypeStruct((N_P,), f32),
            jax.ShapeDtypeStruct((N_P,), f32),
            jax.ShapeDtypeStruct((E_P,), f32),
        ],
        mesh=mesh,
        scratch_types=[
            pltpu.VMEM_SHARED((N_P,), f32),
            pltpu.VMEM_SHARED((N_P,), f32),
            pltpu.VMEM_SHARED((DIM * N_P,), f32),
            pltpu.VMEM_SHARED((N_P,), f32),
            pltpu.VMEM((XN,), f32),
            pltpu.VMEM((EPT,), i32),
            pltpu.VMEM((16, 128), i32),
            pltpu.VMEM((16, 128), i32),
            pltpu.VMEM((EC,), f32),
            pltpu.VMEM((DIM * EC,), f32),
            pltpu.VMEM((NR,), f32),
            pltpu.VMEM((NR,), f32),
            pltpu.VMEM((NR,), f32),
            pltpu.VMEM((1024,), f32),
            pltpu.VMEM((128,), f32),
            pltpu.SemaphoreType.DMA,
        ],
    )


def _mp_body(hwf, src2d, dst2d, normfl, dinv2, zrow,
             o_out, st_out,
             acc_s, tbl, msg, i16s, fnv, i16d,
             abuf, hb, db2, ob, s128, sem):
    cid = lax.axis_index("c")
    sid = lax.axis_index("s")
    wid = cid * NS + sid
    r0 = sid * NR
    eb = sid * EC

    for ls in range(2):
        for f in range(D2):
            pltpu.sync_copy(zrow,
                            acc_s.at[pl.ds((ls * D2 + f) * N_P + r0, NR)])

    def z128(i, c):
        s128[pl.ds(i * VL, VL)] = jnp.zeros((VL,), f32)
        return c
    lax.fori_loop(0, 8, z128, 0)
    plsc.subcore_barrier()

    pltpu.sync_copy(src2d.at[pl.ds(sid * 16, 16)], i16s)
    pltpu.sync_copy(normfl.at[pl.ds(eb, EC)], fnv)
    pltpu.sync_copy(dst2d.at[pl.ds(sid * 16, 16)], i16d)

    for ls in range(2):
        smp = cid * 2 + ls
        pltpu.sync_copy(hwf.at[pl.ds(smp * (D2 * N_P), D2 * N_P)], tbl)

        def bodyf(j, c):
            jf = jnp.broadcast_to(lax.shift_right_logical(j, 3), (VL,))
            posv = _iota() + (j & 7) * VL
            sv = plsc.load_gather(i16s, [jf, posv])
            nv = fnv[pl.ds(j * VL, VL)]
            for f in range(D2):
                val = plsc.load_gather(tbl, [sv + f * N_P]) * nv
                msg[pl.ds(f * EC + j * VL, VL)] = val
            return c
        lax.fori_loop(0, EC // VL, bodyf, 0)

        hs = [pltpu.async_copy(
                  msg.at[pl.ds(f * EC + j * 128, 128)],
                  acc_s.at[pl.ds((ls * D2 + f) * N_P, N_P)].at[i16d.at[j]],
                  sem, add=True)
              for f in range(D2) for j in range(16)]
        for h in hs:
            h.wait()

    plsc.subcore_barrier()

    pltpu.sync_copy(dinv2.at[pl.ds(r0, NR)], db2)
    for ls in range(2):
        smp = cid * 2 + ls
        for f in range(D2):
            pltpu.sync_copy(acc_s.at[pl.ds((ls * D2 + f) * N_P + r0, NR)],
                            abuf.at[pl.ds(f * NR, NR)])
            pltpu.sync_copy(
                hwf.at[pl.ds(smp * (D2 * N_P) + f * N_P + r0, NR)],
                hb.at[pl.ds(f * NR, NR)])

        pairs = []
        for f in range(D2):
            def obody(m, carry):
                s1, s2 = carry
                sl = pl.ds(f * NR + m * VL, VL)
                o = abuf[sl] + db2[pl.ds(m * VL, VL)] * hb[sl]
                ob[sl] = o
                return (s1 + o, s2 + o * o)
            zz = jnp.zeros((VL,), f32)
            s1, s2 = lax.fori_loop(0, NR // VL, obody, (zz, zz))
            pairs.append((f, jnp.sum(s1)))
            pairs.append((D2 + f, jnp.sum(s2)))

        for f in range(D2):
            pltpu.sync_copy(
                ob.at[pl.ds(f * NR, NR)],
                o_out.at[pl.ds(smp * (D2 * N_P) + f * N_P + r0, NR)])
        s128[pl.ds(0, VL)] = _scalar_vec(pairs)
        pltpu.sync_copy(s128, st_out.at[pl.ds((wid * 2 + ls) * 128, 128)])


def _make_mp():
    mesh = plsc.VectorSubcoreMesh(core_axis_name="c", subcore_axis_name="s")
    return pl.kernel(
        _mp_body,
        compiler_params=pltpu.CompilerParams(needs_layout_passes=False),
        out_type=[
            jax.ShapeDtypeStruct((B * D2 * N_P,), f32),
            jax.ShapeDtypeStruct((NW * 2 * 128,), f32),
        ],
        mesh=mesh,
        scratch_types=[
            pltpu.VMEM_SHARED((2 * D2 * N_P,), f32),
            pltpu.VMEM((D2 * N_P,), f32),
            pltpu.VMEM((D2 * EC,), f32),
            pltpu.VMEM((16, 128), i32),
            pltpu.VMEM((EC,), f32),
            pltpu.VMEM((16, 128), i32),
            pltpu.VMEM((D2 * NR,), f32),
            pltpu.VMEM((D2 * NR,), f32),
            pltpu.VMEM((NR,), f32),
            pltpu.VMEM((D2 * NR,), f32),
            pltpu.VMEM((128,), f32),
            pltpu.SemaphoreType.DMA,
        ],
    )


def _t1_body(gb_ref, degv_ref, cpart_ref, w1_ref, w1t_ref, out_ref):
    gbm = gb_ref[...]
    w1t = w1t_ref[...]
    base = jnp.dot(w1t, gbm, preferred_element_type=f32)
    cp = cpart_ref[...].reshape(B, 8, 128)[:, :, 0:DIM]
    centers = cp.sum(axis=1) * (0.5 / E_D)
    u = jnp.dot(centers, w1_ref[...],
                preferred_element_type=f32)
    degv = degv_ref[...]
    mask = (lax.broadcasted_iota(i32, (1, 1, N_P), 2) < N).astype(f32)
    hw = base[None, :, :] - u[:, :, None] * degv[None, None, :]
    out_ref[...] = (hw * mask).reshape(B, D2 * N_P)


def _t1(gb, degv, cpart, w1, w1t):
    return pl.pallas_call(
        _t1_body,
        out_shape=jax.ShapeDtypeStruct((B, D2 * N_P), f32),
    )(gb, degv, cpart, w1, w1t)


def _t2_body(o_ref, st_ref, g_ref, be_ref, wt_ref, out_ref, *, with_w):
    st = st_ref[...]
    s1 = jnp.sum(st[:, 0:D2], axis=0)
    s2 = jnp.sum(st[:, D2:2 * D2], axis=0)
    cnt = float(B * N)
    mu = s1 / cnt
    var = s2 / cnt - mu * mu
    rstd = lax.rsqrt(var + 1e-5)
    g = g_ref[...].reshape(D2)
    be = be_ref[...].reshape(D2)
    o = o_ref[...].reshape(B, D2, N_P)
    a = jnp.maximum((o - mu[None, :, None]) * (rstd * g)[None, :, None]
                    + be[None, :, None], 0.0)
    mask = (lax.broadcasted_iota(i32, (1, 1, N_P), 2) < N).astype(f32)
    a = a * mask
    if with_w:
        wt = wt_ref[...]
        hw = [jnp.dot(wt, a[b], preferred_element_type=f32)
              for b in range(B)]
        out_ref[...] = jnp.stack(hw).reshape(B, D2 * N_P)
    else:
        out_ref[...] = a.reshape(B, D2 * N_P)


def _t2(o, st, g, be, wt, with_w=True):
    return pl.pallas_call(
        functools.partial(_t2_body, with_w=with_w),
        out_shape=jax.ShapeDtypeStruct((B, D2 * N_P), f32),
    )(o, st, g, be, wt)


_KB = 10
_KW = 60000 // _KB


def _mlp_body(x_ref, w1_ref, b1_ref, w2_ref, b2_ref, w3_ref, b3_ref,
              w4_ref, b4_ref, out_ref, acc_ref):
    k = pl.program_id(0)
    xb = x_ref[0]
    partial = jnp.dot(xb, w1_ref[...], preferred_element_type=f32)

    @pl.when(k == 0)
    def _():
        acc_ref[...] = partial

    @pl.when(k > 0)
    def _():
        acc_ref[...] = acc_ref[...] + partial

    @pl.when(k == _KB - 1)
    def _():
        h = jnp.maximum(acc_ref[...] + b1_ref[...], 0.0)
        h = jnp.maximum(jnp.dot(h, w2_ref[...], preferred_element_type=f32)
                        + b2_ref[...], 0.0)
        h = jnp.maximum(jnp.dot(h, w3_ref[...], preferred_element_type=f32)
                        + b3_ref[...], 0.0)
        out_ref[...] = (jnp.dot(h, w4_ref[...], preferred_element_type=f32)
                        + b4_ref[...])


def _mlp(x3, eW1, eb1, eW2, eb2, eW3, eb3, eW4, eb4):
    return pl.pallas_call(
        _mlp_body,
        grid=(_KB,),
        in_specs=[
            pl.BlockSpec((1, 8, _KW), lambda k: (k, 0, 0)),
            pl.BlockSpec((_KW, 256), lambda k: (k, 0)),
            pl.BlockSpec((1, 256), lambda k: (0, 0)),
            pl.BlockSpec((256, 128), lambda k: (0, 0)),
            pl.BlockSpec((1, 128), lambda k: (0, 0)),
            pl.BlockSpec((128, 64), lambda k: (0, 0)),
            pl.BlockSpec((1, 64), lambda k: (0, 0)),
            pl.BlockSpec((64, 32), lambda k: (0, 0)),
            pl.BlockSpec((1, 32), lambda k: (0, 0)),
        ],
        out_specs=pl.BlockSpec((8, 32), lambda k: (0, 0)),
        out_shape=jax.ShapeDtypeStruct((8, 32), f32),
        scratch_shapes=[pltpu.VMEM((8, 256), f32)],
    )(x3, eW1, eb1, eW2, eb2, eW3, eb3, eW4, eb4)


def kernel(x, e, edges, template, W1, b1, W2, b2, W3, b3, g1, be1, g2, be2,
           g3, be3, eW1, eb1, eW2, eb2, eW3, eb3, eW4, eb4):
    padn = jnp.arange(E_P - E_T, dtype=i32) % N
    src2d = jnp.concatenate([edges[:, 0], padn]).reshape(E_P // 128, 128)
    dst2d = jnp.concatenate([edges[:, 1], padn]).reshape(E_P // 128, 128)
    valfl = (jnp.arange(E_P, dtype=i32) < E_T).astype(f32)

    pade = jnp.arange(EP_D - 2 * E_D, dtype=i32) % N
    e_flat = jnp.concatenate(
        [e.reshape(B, 2 * E_D), jnp.tile(pade, (B, 1))], axis=1).reshape(-1)
    xzpad = jnp.zeros((B, XN - DIM * N), f32)
    x_flat = jnp.concatenate(
        [x.reshape(B, DIM * N), xzpad], axis=1).reshape(-1)
    tmpl_flat = jnp.concatenate(
        [template.reshape(DIM * N), jnp.zeros((XN - DIM * N,), f32)])
    zrow = jnp.zeros((NR,), f32)

    s1 = _make_s1()
    degv, gb, cpart, dinv, dinv2, normv = s1(
        src2d, dst2d, valfl, tmpl_flat, e_flat, x_flat, zrow)

    mp = _make_mp()

    hw1 = _t1(gb.reshape(DIM, N_P), degv, cpart.reshape(NW, 128), W1, W1.T)
    o1, st1 = mp(hw1.reshape(-1), src2d, dst2d, normv, dinv2, zrow)
    hw2 = _t2(o1.reshape(B, D2 * N_P), st1.reshape(NW * 2, 128), g1, be1,
              W2.T)
    o2, st2 = mp(hw2.reshape(-1), src2d, dst2d, normv, dinv2, zrow)
    hw3 = _t2(o2.reshape(B, D2 * N_P), st2.reshape(NW * 2, 128), g2, be2,
              W3.T)
    o3, st3 = mp(hw3.reshape(-1), src2d, dst2d, normv, dinv2, zrow)
    a3 = _t2(o3.reshape(B, D2 * N_P), st3.reshape(NW * 2, 128), g3, be3,
             W3.T, with_w=False)

    a3p = a3.reshape(B, D2, N_P)[:, :, :N]
    x2d = a3p.transpose(0, 2, 1).reshape(B, N * D2)
    xp = jnp.concatenate([x2d, jnp.zeros((8 - B, N * D2), f32)], axis=0)
    x3 = xp.reshape(8, _KB, _KW).transpose(1, 0, 2)
    out = _mlp(x3, eW1, eb1.reshape(1, 256), eW2, eb2.reshape(1, 128),
               eW3, eb3.reshape(1, 64), eW4, eb4.reshape(1, 32))
    return out[:B]


# speedup vs baseline: 37.7260x; 1.0009x over previous
import functools

import jax
import jax.numpy as jnp
from jax import lax
from jax.experimental import pallas as pl
from jax.experimental.pallas import tpu as pltpu
from jax.experimental.pallas import tpu_sc as plsc

B = 4
DIM = 3
N = 10000
E_T = 30000
E_D = 30000
D2 = 6

NC = 2
NS = 16
NW = NC * NS
VL = 16

N_P = 10240
NR = N_P // NS
E_P = 32768
EC = E_P // NS
EPT = 7680
EP_D = 8 * EPT
XN = 30720

f32 = jnp.float32
i32 = jnp.int32


def _iota():
    return lax.iota(i32, VL)


def _fastrsqrt(a):
    i = plsc.bitcast(a, i32)
    i = jnp.int32(0x5F3759DF) - lax.shift_right_logical(i, 1)
    y = plsc.bitcast(i, f32)
    for _ in range(3):
        y = y * (1.5 - 0.5 * a * y * y)
    return y


def _scalar_vec(pairs):
    it = _iota()
    v = jnp.zeros((VL,), f32)
    for lane, s in pairs:
        v = jnp.where(it == lane, jnp.full((VL,), s, dtype=f32), v)
    return v


def _s1_body(src2d, dst2d, valfl, tmpl, e_flat, x_flat, zrow,
             gs_o, dinv_o, dinv2_o, normv_o,
             ideg_s, dinv_s, gs_s, cpart_s,
             buf30k, echk, i16a, i16b, fv, msg, cqm, cqp, cbuf, cvm,
             d1, dy, dy2, nbuf, s128, sem):
    cid = lax.axis_index("c")
    sid = lax.axis_index("s")
    wid = cid * NS + sid
    r0 = sid * NR
    ls_mine = lax.shift_right_logical(sid, 3)
    part8 = sid & 7
    eb4 = part8 * 4096

    pltpu.sync_copy(zrow, ideg_s.at[pl.ds(r0, NR)])
    for q in range(2):
        for dim in range(DIM):
            pltpu.sync_copy(zrow,
                            gs_s.at[pl.ds((q * DIM + dim) * N_P + r0, NR)])

    def z128(i, c):
        s128[pl.ds(i * VL, VL)] = jnp.zeros((VL,), f32)
        return c
    lax.fori_loop(0, 8, z128, 0)
    plsc.subcore_barrier()

    pltpu.sync_copy(dst2d.at[pl.ds(part8 * 32, 32)], i16a)
    pltpu.sync_copy(src2d.at[pl.ds(part8 * 32, 32)], i16b)
    pltpu.sync_copy(valfl.at[pl.ds(eb4, 4096)], fv)

    @pl.when(sid < 8)
    def _():
        hs = [pltpu.async_copy(fv.at[pl.ds(j * 128, 128)],
                               ideg_s.at[i16a.at[j]], sem, add=True)
              for j in range(32)]
        for h in hs:
            h.wait()

    pltpu.sync_copy(tmpl, buf30k)

    def tbody(j, c):
        jf = jnp.broadcast_to(lax.shift_right_logical(j, 3), (VL,))
        posv = _iota() + (j & 7) * VL
        sv = plsc.load_gather(i16b, [jf, posv])
        dv = plsc.load_gather(i16a, [jf, posv])
        vf = fv[pl.ds(j * VL, VL)]
        for dim in range(DIM):
            a = plsc.load_gather(buf30k, [sv + dim * N])
            b = plsc.load_gather(buf30k, [dv + dim * N])
            cq = 0.5 * (a + b)
            t2 = 2.0 * (b - a)
            cqm[pl.ds(dim * 4096 + j * VL, VL)] = (cq - t2) * vf
            cqp[pl.ds(dim * 4096 + j * VL, VL)] = (cq + t2) * vf
        return c
    lax.fori_loop(0, 4096 // VL, tbody, 0)

    smp = lax.shift_right_logical(wid, 3)
    part = wid & 7
    pltpu.sync_copy(e_flat.at[pl.ds(smp * EP_D + part * EPT, EPT)], echk)
    pltpu.sync_copy(x_flat.at[pl.ds(smp * XN, XN)], buf30k)

    def cbody(i, carry):
        c0, c1, c2 = carry
        ev = echk[pl.ds(i * VL, VL)]
        posv = _iota() + part * EPT + i * VL
        mb = posv < (2 * E_D)
        g0 = plsc.load_gather(buf30k, [ev])
        g1 = plsc.load_gather(buf30k, [ev + N])
        g2 = plsc.load_gather(buf30k, [ev + 2 * N])
        zero = jnp.zeros((VL,), f32)
        return (c0 + jnp.where(mb, g0, zero),
                c1 + jnp.where(mb, g1, zero),
                c2 + jnp.where(mb, g2, zero))

    z = jnp.zeros((VL,), f32)
    c0, c1, c2 = lax.fori_loop(0, EPT // VL, cbody, (z, z, z))
    s128[pl.ds(0, VL)] = _scalar_vec([(0, jnp.sum(c0)), (1, jnp.sum(c1)),
                                      (2, jnp.sum(c2))])
    pltpu.sync_copy(s128, cpart_s.at[pl.ds(sid * 128, 128)])
    plsc.subcore_barrier()

    pltpu.sync_copy(ideg_s.at[pl.ds(r0, NR)], d1)

    def dbody(i, c):
        a = d1[pl.ds(i * VL, VL)] + 1.0
        y = _fastrsqrt(a)
        dy[pl.ds(i * VL, VL)] = y
        dy2[pl.ds(i * VL, VL)] = y * y
        return c
    lax.fori_loop(0, NR // VL, dbody, 0)

    pltpu.sync_copy(dy, dinv_s.at[pl.ds(r0, NR)])

    @pl.when(cid == 0)
    def _():
        pltpu.sync_copy(dy, dinv_o.at[pl.ds(r0, NR)])
        pltpu.sync_copy(dy2, dinv2_o.at[pl.ds(r0, NR)])

    plsc.subcore_barrier()

    pltpu.sync_copy(cpart_s.at[pl.ds(ls_mine * 8 * 128, 1024)], cbuf)
    cv = jnp.zeros((VL,), f32)
    for r in range(8):
        cv = cv + cbuf[pl.ds(r * 128, VL)]
    cvs = cv * (0.5 / E_D)
    it16 = _iota()
    z16 = jnp.zeros((VL,), f32)

    for pre, irows in ((cqm, i16b), (cqp, i16a)):
        for dim in range(DIM):
            cB = jnp.full((VL,), jnp.sum(jnp.where(it16 == dim, cvs, z16)),
                          dtype=f32)

            def mbody(j, c, _pre=pre, _dim=dim, _cB=cB):
                sl = pl.ds(_dim * 4096 + j * VL, VL)
                vf = fv[pl.ds(j * VL, VL)]
                msg[sl] = _pre[sl] - _cB * vf
                return c
            lax.fori_loop(0, 4096 // VL, mbody, 0)
        hs = [pltpu.async_copy(
                  msg.at[pl.ds(dim * 4096 + j * 128, 128)],
                  gs_s.at[pl.ds((ls_mine * DIM + dim) * N_P, N_P)]
                      .at[irows.at[j]],
                  sem, add=True)
              for dim in range(DIM) for j in range(32)]
        for h in hs:
            h.wait()

    pltpu.sync_copy(dinv_s, buf30k.at[pl.ds(0, N_P)])
    nb = wid * 1024
    pltpu.sync_copy(src2d.at[pl.ds(wid * 8, 8)], i16b.at[pl.ds(0, 8)])
    pltpu.sync_copy(dst2d.at[pl.ds(wid * 8, 8)], i16a.at[pl.ds(0, 8)])
    pltpu.sync_copy(valfl.at[pl.ds(nb, 1024)], fv.at[pl.ds(0, 1024)])

    def nbody(j, c):
        jf = jnp.broadcast_to(lax.shift_right_logical(j, 3), (VL,))
        posv = _iota() + (j & 7) * VL
        sv = plsc.load_gather(i16b, [jf, posv])
        dv = plsc.load_gather(i16a, [jf, posv])
        vf = fv[pl.ds(j * VL, VL)]
        nv = plsc.load_gather(buf30k, [sv]) * plsc.load_gather(buf30k, [dv])
        nbuf[pl.ds(j * VL, VL)] = nv * vf
        return c
    lax.fori_loop(0, 1024 // VL, nbody, 0)
    pltpu.sync_copy(nbuf, normv_o.at[pl.ds(nb, 1024)])
    plsc.subcore_barrier()

    for q in range(2):
        for dim in range(DIM):
            pltpu.sync_copy(gs_s.at[pl.ds((q * DIM + dim) * N_P + r0, NR)],
                            d1)
            pltpu.sync_copy(
                d1,
                gs_o.at[pl.ds(((cid * 2 + q) * DIM + dim) * N_P + r0, NR)])


def _make_s1():
    mesh = plsc.VectorSubcoreMesh(core_axis_name="c", subcore_axis_name="s")
    return pl.kernel(
        _s1_body,
        compiler_params=pltpu.CompilerParams(needs_layout_passes=False),
        out_type=[
            jax.ShapeDtypeStruct((B * DIM * N_P,), f32),
            jax.ShapeDtypeStruct((N_P,), f32),
            jax.ShapeDtypeStruct((N_P,), f32),
            jax.ShapeDtypeStruct((E_P,), f32),
        ],
        mesh=mesh,
        scratch_types=[
            pltpu.VMEM_SHARED((N_P,), f32),
            pltpu.VMEM_SHARED((N_P,), f32),
            pltpu.VMEM_SHARED((2 * DIM * N_P,), f32),
            pltpu.VMEM_SHARED((NS * 128,), f32),
            pltpu.VMEM((XN,), f32),
            pltpu.VMEM((EPT,), i32),
            pltpu.VMEM((32, 128), i32),
            pltpu.VMEM((32, 128), i32),
            pltpu.VMEM((4096,), f32),
            pltpu.VMEM((DIM * 4096,), f32),
            pltpu.VMEM((DIM * 4096,), f32),
            pltpu.VMEM((DIM * 4096,), f32),
            pltpu.VMEM((1024,), f32),
            pltpu.VMEM((VL,), f32),
            pltpu.VMEM((NR,), f32),
            pltpu.VMEM((NR,), f32),
            pltpu.VMEM((NR,), f32),
            pltpu.VMEM((1024,), f32),
            pltpu.VMEM((128,), f32),
            pltpu.SemaphoreType.DMA,
        ],
    )


def _mp_body(hwf, src2d, dst2d, normfl, dinv2, zrow,
             o_out, st_out,
             acc_s, tbl, msg, i16s, fnv, i16d,
             abuf, hb, db2, ob, s128, sem):
    cid = lax.axis_index("c")
    sid = lax.axis_index("s")
    wid = cid * NS + sid
    r0 = sid * NR
    ls_mine = lax.shift_right_logical(sid, 3)
    smp = cid * 2 + ls_mine
    part8 = sid & 7
    eb4 = part8 * 4096

    for q in range(2):
        for f in range(D2):
            pltpu.sync_copy(zrow,
                            acc_s.at[pl.ds((q * D2 + f) * N_P + r0, NR)])

    def z128(i, c):
        s128[pl.ds(i * VL, VL)] = jnp.zeros((VL,), f32)
        return c
    lax.fori_loop(0, 8, z128, 0)
    plsc.subcore_barrier()

    pltpu.sync_copy(src2d.at[pl.ds(part8 * 32, 32)], i16s)
    pltpu.sync_copy(dst2d.at[pl.ds(part8 * 32, 32)], i16d)
    pltpu.sync_copy(normfl.at[pl.ds(eb4, 4096)], fnv)
    pltpu.sync_copy(hwf.at[pl.ds(smp * (D2 * N_P), D2 * N_P)], tbl)

    def bodyf(j, c):
        jf = jnp.broadcast_to(lax.shift_right_logical(j, 3), (VL,))
        posv = _iota() + (j & 7) * VL
        sv = plsc.load_gather(i16s, [jf, posv])
        nv = fnv[pl.ds(j * VL, VL)]
        for f in range(D2):
            val = plsc.load_gather(tbl, [sv + f * N_P]) * nv
            msg[pl.ds(f * 4096 + j * VL, VL)] = val
        return c
    lax.fori_loop(0, 4096 // VL, bodyf, 0)

    for half in range(2):
        hs = [pltpu.async_copy(
                  msg.at[pl.ds(f * 4096 + j * 128, 128)],
                  acc_s.at[pl.ds((ls_mine * D2 + f) * N_P, N_P)]
                       .at[i16d.at[j]],
                  sem, add=True)
              for f in range(half * 3, half * 3 + 3) for j in range(32)]
        for h in hs:
            h.wait()

    plsc.subcore_barrier()

    pltpu.sync_copy(dinv2.at[pl.ds(r0, NR)], db2)
    for ls in range(2):
        smp2 = cid * 2 + ls
        for f in range(D2):
            pltpu.sync_copy(acc_s.at[pl.ds((ls * D2 + f) * N_P + r0, NR)],
                            abuf.at[pl.ds(f * NR, NR)])
            pltpu.sync_copy(
                hwf.at[pl.ds(smp2 * (D2 * N_P) + f * N_P + r0, NR)],
                hb.at[pl.ds(f * NR, NR)])

        pairs = []
        for f in range(D2):
            def obody(m, carry):
                s1, s2 = carry
                sl = pl.ds(f * NR + m * VL, VL)
                o = abuf[sl] + db2[pl.ds(m * VL, VL)] * hb[sl]
                ob[sl] = o
                return (s1 + o, s2 + o * o)
            zz = jnp.zeros((VL,), f32)
            s1, s2 = lax.fori_loop(0, NR // VL, obody, (zz, zz))
            pairs.append((f, jnp.sum(s1)))
            pairs.append((D2 + f, jnp.sum(s2)))

        for f in range(D2):
            pltpu.sync_copy(
                ob.at[pl.ds(f * NR, NR)],
                o_out.at[pl.ds(smp2 * (D2 * N_P) + f * N_P + r0, NR)])
        s128[pl.ds(0, VL)] = _scalar_vec(pairs)
        pltpu.sync_copy(s128, st_out.at[pl.ds((wid * 2 + ls) * 128, 128)])


def _make_mp():
    mesh = plsc.VectorSubcoreMesh(core_axis_name="c", subcore_axis_name="s")
    return pl.kernel(
        _mp_body,
        compiler_params=pltpu.CompilerParams(needs_layout_passes=False),
        out_type=[
            jax.ShapeDtypeStruct((B * D2 * N_P,), f32),
            jax.ShapeDtypeStruct((NW * 2 * 128,), f32),
        ],
        mesh=mesh,
        scratch_types=[
            pltpu.VMEM_SHARED((2 * D2 * N_P,), f32),
            pltpu.VMEM((D2 * N_P,), f32),
            pltpu.VMEM((D2 * 4096,), f32),
            pltpu.VMEM((32, 128), i32),
            pltpu.VMEM((4096,), f32),
            pltpu.VMEM((32, 128), i32),
            pltpu.VMEM((D2 * NR,), f32),
            pltpu.VMEM((D2 * NR,), f32),
            pltpu.VMEM((NR,), f32),
            pltpu.VMEM((D2 * NR,), f32),
            pltpu.VMEM((128,), f32),
            pltpu.SemaphoreType.DMA,
        ],
    )


def _t1_body(gs_ref, w1t_ref, out_ref):
    gs = gs_ref[...]
    w1t = w1t_ref[...]
    hw = [jnp.dot(w1t, gs[b], preferred_element_type=f32)
          for b in range(B)]
    hw = jnp.stack(hw)
    mask = (lax.broadcasted_iota(i32, (1, 1, N_P), 2) < N).astype(f32)
    out_ref[...] = (hw * mask).reshape(B, D2 * N_P)


def _t1(gs3, w1t):
    return pl.pallas_call(
        _t1_body,
        out_shape=jax.ShapeDtypeStruct((B, D2 * N_P), f32),
    )(gs3, w1t)


def _t2_body(o_ref, st_ref, g_ref, be_ref, wt_ref, out_ref, *, with_w):
    st = st_ref[...]
    s1 = jnp.sum(st[:, 0:D2], axis=0)
    s2 = jnp.sum(st[:, D2:2 * D2], axis=0)
    cnt = float(B * N)
    mu = s1 / cnt
    var = s2 / cnt - mu * mu
    rstd = lax.rsqrt(var + 1e-5)
    g = g_ref[...].reshape(D2)
    be = be_ref[...].reshape(D2)
    o = o_ref[...].reshape(B, D2, N_P)
    a = jnp.maximum((o - mu[None, :, None]) * (rstd * g)[None, :, None]
                    + be[None, :, None], 0.0)
    mask = (lax.broadcasted_iota(i32, (1, 1, N_P), 2) < N).astype(f32)
    a = a * mask
    if with_w:
        wt = wt_ref[...]
        hw = [jnp.dot(wt, a[b], preferred_element_type=f32)
              for b in range(B)]
        out_ref[...] = jnp.stack(hw).reshape(B, D2 * N_P)
    else:
        out_ref[...] = a.reshape(B, D2 * N_P)


def _t2(o, st, g, be, wt, with_w=True):
    return pl.pallas_call(
        functools.partial(_t2_body, with_w=with_w),
        out_shape=jax.ShapeDtypeStruct((B, D2 * N_P), f32),
    )(o, st, g, be, wt)


_KB = 10
_KW = 60000 // _KB


def _mlp_body(x_ref, w1_ref, b1_ref, w2_ref, b2_ref, w3_ref, b3_ref,
              w4_ref, b4_ref, out_ref, acc_ref):
    k = pl.program_id(0)
    xb = x_ref[0]
    partial = jnp.dot(xb, w1_ref[...], preferred_element_type=f32)

    @pl.when(k == 0)
    def _():
        acc_ref[...] = partial

    @pl.when(k > 0)
    def _():
        acc_ref[...] = acc_ref[...] + partial

    @pl.when(k == _KB - 1)
    def _():
        h = jnp.maximum(acc_ref[...] + b1_ref[...], 0.0)
        h = jnp.maximum(jnp.dot(h, w2_ref[...], preferred_element_type=f32)
                        + b2_ref[...], 0.0)
        h = jnp.maximum(jnp.dot(h, w3_ref[...], preferred_element_type=f32)
                        + b3_ref[...], 0.0)
        out_ref[...] = (jnp.dot(h, w4_ref[...], preferred_element_type=f32)
                        + b4_ref[...])


def _mlp(x3, eW1, eb1, eW2, eb2, eW3, eb3, eW4, eb4):
    return pl.pallas_call(
        _mlp_body,
        grid=(_KB,),
        in_specs=[
            pl.BlockSpec((1, 8, _KW), lambda k: (k, 0, 0)),
            pl.BlockSpec((_KW, 256), lambda k: (k, 0)),
            pl.BlockSpec((1, 256), lambda k: (0, 0)),
            pl.BlockSpec((256, 128), lambda k: (0, 0)),
            pl.BlockSpec((1, 128), lambda k: (0, 0)),
            pl.BlockSpec((128, 64), lambda k: (0, 0)),
            pl.BlockSpec((1, 64), lambda k: (0, 0)),
            pl.BlockSpec((64, 32), lambda k: (0, 0)),
            pl.BlockSpec((1, 32), lambda k: (0, 0)),
        ],
        out_specs=pl.BlockSpec((8, 32), lambda k: (0, 0)),
        out_shape=jax.ShapeDtypeStruct((8, 32), f32),
        scratch_shapes=[pltpu.VMEM((8, 256), f32)],
    )(x3, eW1, eb1, eW2, eb2, eW3, eb3, eW4, eb4)


def kernel(x, e, edges, template, W1, b1, W2, b2, W3, b3, g1, be1, g2, be2,
           g3, be3, eW1, eb1, eW2, eb2, eW3, eb3, eW4, eb4):
    padn = jnp.arange(E_P - E_T, dtype=i32) % N
    src2d = jnp.concatenate([edges[:, 0], padn]).reshape(E_P // 128, 128)
    dst2d = jnp.concatenate([edges[:, 1], padn]).reshape(E_P // 128, 128)
    valfl = (jnp.arange(E_P, dtype=i32) < E_T).astype(f32)

    pade = jnp.arange(EP_D - 2 * E_D, dtype=i32) % N
    e_flat = jnp.concatenate(
        [e.reshape(B, 2 * E_D), jnp.tile(pade, (B, 1))], axis=1).reshape(-1)
    xzpad = jnp.zeros((B, XN - DIM * N), f32)
    x_flat = jnp.concatenate(
        [x.reshape(B, DIM * N), xzpad], axis=1).reshape(-1)
    tmpl_flat = jnp.concatenate(
        [template.reshape(DIM * N), jnp.zeros((XN - DIM * N,), f32)])
    zrow = jnp.zeros((NR,), f32)

    s1 = _make_s1()
    gs, dinv, dinv2, normv = s1(
        src2d, dst2d, valfl, tmpl_flat, e_flat, x_flat, zrow)

    mp = _make_mp()

    hw1 = _t1(gs.reshape(B, DIM, N_P), W1.T)
    o1, st1 = mp(hw1.reshape(-1), src2d, dst2d, normv, dinv2, zrow)
    hw2 = _t2(o1.reshape(B, D2 * N_P), st1.reshape(NW * 2, 128), g1, be1,
              W2.T)
    o2, st2 = mp(hw2.reshape(-1), src2d, dst2d, normv, dinv2, zrow)
    hw3 = _t2(o2.reshape(B, D2 * N_P), st2.reshape(NW * 2, 128), g2, be2,
              W3.T)
    o3, st3 = mp(hw3.reshape(-1), src2d, dst2d, normv, dinv2, zrow)
    a3 = _t2(o3.reshape(B, D2 * N_P), st3.reshape(NW * 2, 128), g3, be3,
             W3.T, with_w=False)

    a3p = a3.reshape(B, D2, N_P)[:, :, :N]
    x2d = a3p.transpose(0, 2, 1).reshape(B, N * D2)
    xp = jnp.concatenate([x2d, jnp.zeros((8 - B, N * D2), f32)], axis=0)
    x3 = xp.reshape(8, _KB, _KW).transpose(1, 0, 2)
    out = _mlp(x3, eW1, eb1.reshape(1, 256), eW2, eb2.reshape(1, 128),
               eW3, eb3.reshape(1, 64), eW4, eb4.reshape(1, 32))
    return out[:B]
